# baseline (speedup 1.0000x reference)
import functools

import jax
import jax.numpy as jnp
from jax import lax
from jax.experimental import pallas as pl
from jax.experimental.pallas import tpu as pltpu
from jax.experimental.pallas import tpu_sc as plsc

D = 16
L = 16
NC = 2
NS = 16
NW = NC * NS
W = 1152
NWIN = 868
COVER = NWIN * W
TAIL = 64
NJ = 28
MCAP = 768
SEGCAP = 128
IDX_SENTINEL = 1 << 20


def _w_of(ids):
    return ((ids >> 7) * 7282) >> 16


@functools.lru_cache(maxsize=None)
def _build_k1(B: int):
    mesh = plsc.VectorSubcoreMesh(core_axis_name="c", subcore_axis_name="s")
    stage_n = B * D + 128

    @functools.partial(
        pl.kernel,
        mesh=mesh,
        out_type=[
            jax.ShapeDtypeStruct((stage_n,), jnp.float32),
            jax.ShapeDtypeStruct((stage_n,), jnp.float32),
        ],
        compiler_params=pltpu.CompilerParams(needs_layout_passes=False),
        scratch_types=[
            pltpu.VMEM((4096,), jnp.int32),
            pltpu.VMEM((MCAP,), jnp.int32),
            pltpu.VMEM((MCAP,), jnp.int32),
            pltpu.VMEM((MCAP,), jnp.int32),
            pltpu.VMEM((MCAP,), jnp.int32),
            pltpu.VMEM((8, W), jnp.float32),
            pltpu.VMEM((8, W), jnp.float32),
            pltpu.VMEM((8, W), jnp.float32),
            pltpu.VMEM((8, W), jnp.float32),
            pltpu.VMEM((8, W), jnp.float32),
            pltpu.VMEM((8, W), jnp.float32),
            pltpu.VMEM((8, W), jnp.float32),
            pltpu.VMEM((8, W), jnp.float32),
            pltpu.VMEM((D, 128), jnp.float32),
            pltpu.VMEM((D, 128), jnp.float32),
            pltpu.VMEM((SEGCAP,), jnp.int32),
            pltpu.VMEM((SEGCAP,), jnp.int32),
            pltpu.VMEM((D, MCAP), jnp.float32),
            pltpu.VMEM((D, MCAP), jnp.float32),
            pltpu.VMEM((MCAP,), jnp.int32),
            pltpu.VMEM((MCAP,), jnp.int32),
            pltpu.VMEM((D, 128), jnp.int32),
            pltpu.SemaphoreType.DMA,
            pltpu.SemaphoreType.DMA,
            pltpu.SemaphoreType.DMA,
            pltpu.SemaphoreType.DMA,
        ],
    )
    def k1(users_hbm, items_hbm, utt_hbm, itt_hbm, ut_tail_hbm, it_tail_hbm,
           ustage_hbm, istage_hbm,
           scanbuf, umpos, umid, impos, imid,
           u0a, u1a, i0a, i1a, u0b, u1b, i0b, i1b,
           utail, itail, segpos, segloc,
           upend, ipend, uppos, ippos, idxrows,
           sem0, sem1, scsem, msem):
        wid = lax.axis_index("s") * NC + lax.axis_index("c")
        iota = lax.iota(jnp.int32, L)

        ubufs = ((u0a, u1a), (u0b, u1b))
        ibufs = ((i0a, i1a), (i0b, i1b))
        sems = (sem0, sem1)

        big = jnp.full((L,), IDX_SENTINEL, jnp.int32)
        sinkpos = jnp.full((L,), B, jnp.int32)

        def prefill(g, _):
            umid[pl.ds(g * L, L)] = big
            imid[pl.ds(g * L, L)] = big
            umpos[pl.ds(g * L, L)] = sinkpos
            impos[pl.ds(g * L, L)] = sinkpos
            uppos[pl.ds(g * L, L)] = sinkpos
            ippos[pl.ds(g * L, L)] = sinkpos
            return 0

        lax.fori_loop(0, MCAP // L, prefill, 0)
        segpos[pl.ds(0, L)] = sinkpos
        segloc[pl.ds(0, L)] = jnp.zeros((L,), jnp.int32)

        def scan(idx_hbm, mpos, mid):
            cnt = jnp.int32(0)
            for ch in range(B // 4096):
                pltpu.sync_copy(idx_hbm.at[pl.ds(ch * 4096, 4096)], scanbuf)

                def scan_g(g, cnt):
                    ids = scanbuf[pl.ds(g * L, L)]
                    w = _w_of(ids)
                    mask = (w & 31) == wid
                    slots = cnt + plsc.cumsum(mask.astype(jnp.int32)) - 1
                    plsc.store_scatter(
                        mpos, [slots], ch * 4096 + g * L + iota, mask=mask)
                    plsc.store_scatter(mid, [slots], ids, mask=mask)
                    cnt = cnt + plsc.all_reduce_population_count(mask)[0]
                    return jnp.minimum(cnt, MCAP - L)

                cnt = lax.fori_loop(0, 4096 // L, scan_g, cnt)
            return cnt

        ucnt = scan(users_hbm, umpos, umid)
        icnt = scan(items_hbm, impos, imid)

        def fire(j, slot):
            k = jnp.int32(wid) + 32 * j
            k = jnp.where(k < NWIN, k, 0)
            off = pl.multiple_of(k * W, 128)
            cps = []
            for tr in range(2):
                cps.append(pltpu.async_copy(
                    utt_hbm.at[pl.ds(tr * 8, 8), pl.ds(off, W)],
                    ubufs[slot][tr], sems[slot]))
                cps.append(pltpu.async_copy(
                    itt_hbm.at[pl.ds(tr * 8, 8), pl.ds(off, W)],
                    ibufs[slot][tr], sems[slot]))
            return cps

        def extract(k, bufs, mpos, mid, pend, ppos, pcnt):
            def seg_g(g, scnt):
                ids = mid[pl.ds(g * L, L)]
                mask = _w_of(ids) == k
                slots = scnt + plsc.cumsum(mask.astype(jnp.int32)) - 1
                plsc.store_scatter(
                    segpos, [slots], mpos[pl.ds(g * L, L)], mask=mask)
                plsc.store_scatter(segloc, [slots], ids - k * W, mask=mask)
                scnt = scnt + plsc.all_reduce_population_count(mask)[0]
                return jnp.minimum(scnt, SEGCAP - L)

            scnt = lax.fori_loop(0, MCAP // L, seg_g, jnp.int32(L))

            def ext_g(g, pc):
                m = scnt - L - g * L
                gmask = iota < m
                loc = jnp.where(gmask, segloc[pl.ds(L + g * L, L)], 0)
                pos = segpos[pl.ds(L + g * L, L)]
                npc = pc + jnp.minimum(jnp.maximum(m, 0), L)
                npc = jnp.minimum(npc, MCAP - L)

                slots = pc + iota

                @pl.when(m > 0)
                def _():
                    for c in range(D):
                        src = bufs[c // 8] if len(bufs) == 2 else bufs[0]
                        crow = c % 8 if len(bufs) == 2 else c
                        vals = plsc.load_gather(
                            src, [jnp.full((L,), crow, jnp.int32), loc])
                        plsc.store_scatter(
                            pend, [jnp.full((L,), c, jnp.int32), slots],
                            vals, mask=gmask)
                    plsc.store_scatter(
                        ppos, [slots], jnp.where(gmask, pos, B), mask=gmask)
                return npc

            return lax.fori_loop(0, SEGCAP // L, ext_g, pcnt)

        def drain(slot):
            for tr in range(2):
                pltpu.make_async_copy(
                    utt_hbm.at[pl.ds(tr * 8, 8), pl.ds(0, W)],
                    ubufs[slot][tr], sems[slot]).wait()
                pltpu.make_async_copy(
                    itt_hbm.at[pl.ds(tr * 8, 8), pl.ds(0, W)],
                    ibufs[slot][tr], sems[slot]).wait()

        fire(jnp.int32(0), 0)
        fire(jnp.int32(1), 1)

        def window_pair(jj, carry):
            upc, ipc = carry
            j0 = 2 * jj
            k0 = jnp.int32(wid) + 32 * j0
            k1_ = k0 + 32
            drain(0)
            upc = extract(k0, ubufs[0], umpos, umid, upend, uppos, upc)
            ipc = extract(k0, ibufs[0], impos, imid, ipend, ippos, ipc)
            fire(j0 + 2, 0)
            drain(1)
            upc = extract(k1_, ubufs[1], umpos, umid, upend, uppos, upc)
            ipc = extract(k1_, ibufs[1], impos, imid, ipend, ippos, ipc)
            fire(j0 + 3, 1)
            return upc, ipc

        upc, ipc = lax.fori_loop(
            0, NJ // 2, window_pair, (jnp.int32(0), jnp.int32(0)))
        drain(0)
        drain(1)

        pltpu.sync_copy(ut_tail_hbm, utail)
        pltpu.sync_copy(it_tail_hbm, itail)
        upc = extract(jnp.int32(NWIN), (utail,), umpos, umid,
                      upend, uppos, upc)
        ipc = extract(jnp.int32(NWIN), (itail,), impos, imid,
                      ipend, ippos, ipc)

        def flush(pend, ppos, stage_hbm):
            def flush_ch(ch, _):
                for c in range(D):
                    for g in range(128 // L):
                        p = ppos[pl.ds(ch * 128 + g * L, L)]
                        idxrows[c, pl.ds(g * L, L)] = p * D + c
                waits = []
                for c in range(D):
                    waits.append(pltpu.async_copy(
                        pend.at[c, pl.ds(ch * 128, 128)],
                        stage_hbm.at[idxrows.at[c]], scsem))
                for cp in waits:
                    cp.wait()
                return 0

            lax.fori_loop(0, MCAP // 128, flush_ch, 0)

        flush(upend, uppos, ustage_hbm)
        flush(ipend, ippos, istage_hbm)

    return k1


@functools.lru_cache(maxsize=None)
def _build_k2(B: int):
    b_per_w = B // NW
    n_blocks = b_per_w // L
    mesh = plsc.VectorSubcoreMesh(core_axis_name="c", subcore_axis_name="s")
    stage_n = B * D + 128

    @functools.partial(
        pl.kernel,
        mesh=mesh,
        out_type=jax.ShapeDtypeStruct((B,), jnp.float32),
        compiler_params=pltpu.CompilerParams(needs_layout_passes=False),
        scratch_types=[
            pltpu.VMEM((b_per_w // 128, 128), jnp.int32),
            pltpu.VMEM((b_per_w // 128, 128), jnp.int32),
            pltpu.VMEM((b_per_w * D,), jnp.float32),
            pltpu.VMEM((b_per_w * D,), jnp.float32),
            pltpu.VMEM((b_per_w,), jnp.float32),
            pltpu.VMEM((b_per_w,), jnp.float32),
            pltpu.VMEM((b_per_w,), jnp.float32),
            pltpu.SemaphoreType.DMA,
        ],
    )
    def k2(users_hbm, items_hbm, ub_hbm, ib_hbm, ustage_hbm, istage_hbm,
           out_hbm, uidx, iidx, uflat, iflat, ubias, ibias, outv, sem):
        wid = lax.axis_index("s") * NC + lax.axis_index("c")
        base = wid * b_per_w

        cps = [
            pltpu.async_copy(
                ustage_hbm.at[pl.ds(base * D, b_per_w * D)], uflat, sem),
            pltpu.async_copy(
                istage_hbm.at[pl.ds(base * D, b_per_w * D)], iflat, sem),
        ]
        for c in range(b_per_w // 128):
            pltpu.sync_copy(users_hbm.at[pl.ds(base + c * 128, 128)],
                            uidx.at[c])
            pltpu.sync_copy(items_hbm.at[pl.ds(base + c * 128, 128)],
                            iidx.at[c])
            cps.append(pltpu.async_copy(
                ub_hbm.at[uidx.at[c]], ubias.at[pl.ds(c * 128, 128)], sem))
            cps.append(pltpu.async_copy(
                ib_hbm.at[iidx.at[c]], ibias.at[pl.ds(c * 128, 128)], sem))
        for cp in cps:
            cp.wait()

        iota = lax.iota(jnp.int32, L)
        for b in range(n_blocks):
            rows = (b * L * D) + iota * D
            acc = ubias[pl.ds(b * L, L)] + ibias[pl.ds(b * L, L)]
            for j in range(D):
                u = plsc.load_gather(uflat, [rows + j])
                v = plsc.load_gather(iflat, [rows + j])
                acc = acc + u * v
            outv[pl.ds(b * L, L)] = acc

        pltpu.sync_copy(outv, out_hbm.at[pl.ds(base, b_per_w)])

    return k2


def kernel(users, items, user_table, item_table, user_bias_table, item_bias_table):
    B = users.shape[0]
    users = users.astype(jnp.int32)
    items = items.astype(jnp.int32)
    utt = user_table.T
    itt = item_table.T
    ut_tail = jnp.pad(utt[:, COVER:], ((0, 0), (0, 128 - TAIL)))
    it_tail = jnp.pad(itt[:, COVER:], ((0, 0), (0, 128 - TAIL)))
    ub1 = user_bias_table.reshape(-1)
    ib1 = item_bias_table.reshape(-1)

    ustage, istage = _build_k1(B)(users, items, utt, itt, ut_tail, it_tail)
    out = _build_k2(B)(users, items, ub1, ib1, ustage, istage)
    return out.reshape(B, 1)


# speedup vs baseline: 207.3507x; 207.3507x over previous
import functools

import jax
import jax.numpy as jnp
from jax import lax
from jax.experimental import pallas as pl
from jax.experimental.pallas import tpu as pltpu
from jax.experimental.pallas import tpu_sc as plsc

D = 16
L = 16
NC = 2
NS = 16
NW = NC * NS
W = 1152
NWIN = 868
COVER = NWIN * W
TAIL = 64
NJ = 28
MCAP = 768
SEGCAP = 128
IDX_SENTINEL = 1 << 20


def _w_of(ids):
    return ((ids >> 7) * 7282) >> 16


@functools.lru_cache(maxsize=None)
def _build_k1(B: int):
    mesh = plsc.VectorSubcoreMesh(core_axis_name="c", subcore_axis_name="s")
    stage_n = B * D + 128

    @functools.partial(
        pl.kernel,
        mesh=mesh,
        out_type=[
            jax.ShapeDtypeStruct((NW * D * MCAP,), jnp.float32),
            jax.ShapeDtypeStruct((NW * D * MCAP,), jnp.float32),
            jax.ShapeDtypeStruct((B + 128,), jnp.int32),
            jax.ShapeDtypeStruct((B + 128,), jnp.int32),
            jax.ShapeDtypeStruct((B + 128,), jnp.int32),
            jax.ShapeDtypeStruct((B + 128,), jnp.int32),
        ],
        compiler_params=pltpu.CompilerParams(needs_layout_passes=False),
        scratch_types=[
            pltpu.VMEM((4096,), jnp.int32),
            pltpu.VMEM((MCAP,), jnp.int32),
            pltpu.VMEM((MCAP,), jnp.int32),
            pltpu.VMEM((MCAP,), jnp.int32),
            pltpu.VMEM((MCAP,), jnp.int32),
            pltpu.VMEM((8, W), jnp.float32),
            pltpu.VMEM((8, W), jnp.float32),
            pltpu.VMEM((8, W), jnp.float32),
            pltpu.VMEM((8, W), jnp.float32),
            pltpu.VMEM((8, W), jnp.float32),
            pltpu.VMEM((8, W), jnp.float32),
            pltpu.VMEM((8, W), jnp.float32),
            pltpu.VMEM((8, W), jnp.float32),
            pltpu.VMEM((D, 128), jnp.float32),
            pltpu.VMEM((D, 128), jnp.float32),
            pltpu.VMEM((SEGCAP,), jnp.int32),
            pltpu.VMEM((SEGCAP,), jnp.int32),
            pltpu.VMEM((D, MCAP), jnp.float32),
            pltpu.VMEM((D, MCAP), jnp.float32),
            pltpu.VMEM((MCAP,), jnp.int32),
            pltpu.VMEM((MCAP,), jnp.int32),
            pltpu.VMEM((D, 128), jnp.int32),
            pltpu.VMEM_SHARED((B + 128,), jnp.int32),
            pltpu.VMEM_SHARED((B + 128,), jnp.int32),
            pltpu.VMEM((4096,), jnp.int32),
            pltpu.SemaphoreType.DMA,
            pltpu.SemaphoreType.DMA,
            pltpu.SemaphoreType.DMA,
            pltpu.SemaphoreType.DMA,
        ],
    )
    def k1(users_hbm, items_hbm, utt_hbm, itt_hbm, ut_tail_hbm, it_tail_hbm,
           urows_hbm, irows_hbm, pmu0_hbm, pmu1_hbm, pmi0_hbm, pmi1_hbm,
           scanbuf, umpos, umid, impos, imid,
           u0a, u1a, i0a, i1a, u0b, u1b, i0b, i1b,
           utail, itail, segpos, segloc,
           upend, ipend, uppos, ippos, idxrows,
           upmap_sp, ipmap_sp, bounce,
           sem0, sem1, scsem, msem):
        cid = lax.axis_index("c")
        sid = lax.axis_index("s")
        wid = sid * NC + cid
        iota = lax.iota(jnp.int32, L)

        ubufs = ((u0a, u1a), (u0b, u1b))
        ibufs = ((i0a, i1a), (i0b, i1b))
        sems = (sem0, sem1)

        neg = jnp.full((L,), -1, jnp.int32)

        def fill_neg(g, _):
            bounce[pl.ds(g * L, L)] = neg
            return 0

        lax.fori_loop(0, 4096 // L, fill_neg, 0)

        @pl.when(sid == 0)
        def _():
            for c4 in range(4):
                pltpu.sync_copy(bounce, upmap_sp.at[pl.ds(c4 * 4096, 4096)])
                pltpu.sync_copy(bounce, ipmap_sp.at[pl.ds(c4 * 4096, 4096)])
            pltpu.sync_copy(bounce.at[pl.ds(0, 128)],
                            upmap_sp.at[pl.ds(4 * 4096, 128)])
            pltpu.sync_copy(bounce.at[pl.ds(0, 128)],
                            ipmap_sp.at[pl.ds(4 * 4096, 128)])

        plsc.subcore_barrier()

        big = jnp.full((L,), IDX_SENTINEL, jnp.int32)
        sinkpos = jnp.full((L,), B, jnp.int32)

        def prefill(g, _):
            umid[pl.ds(g * L, L)] = big
            imid[pl.ds(g * L, L)] = big
            umpos[pl.ds(g * L, L)] = sinkpos
            impos[pl.ds(g * L, L)] = sinkpos
            uppos[pl.ds(g * L, L)] = sinkpos
            ippos[pl.ds(g * L, L)] = sinkpos
            return 0

        lax.fori_loop(0, MCAP // L, prefill, 0)
        segpos[pl.ds(0, L)] = sinkpos
        segloc[pl.ds(0, L)] = jnp.zeros((L,), jnp.int32)

        def scan(idx_hbm, mpos, mid):
            cnt = jnp.int32(0)
            for ch in range(B // 4096):
                pltpu.sync_copy(idx_hbm.at[pl.ds(ch * 4096, 4096)], scanbuf)

                def scan_g(g, cnt):
                    ids = scanbuf[pl.ds(g * L, L)]
                    w = _w_of(ids)
                    mask = (w & 31) == wid
                    slots = cnt + plsc.cumsum(mask.astype(jnp.int32)) - 1
                    plsc.store_scatter(
                        mpos, [slots], ch * 4096 + g * L + iota, mask=mask)
                    plsc.store_scatter(mid, [slots], ids, mask=mask)
                    cnt = cnt + plsc.all_reduce_population_count(mask)[0]
                    return jnp.minimum(cnt, MCAP - L)

                cnt = lax.fori_loop(0, 4096 // L, scan_g, cnt)
            return cnt

        ucnt = scan(users_hbm, umpos, umid)
        icnt = scan(items_hbm, impos, imid)

        def fire(j, slot):
            k = jnp.int32(wid) + 32 * j
            k = jnp.where(k < NWIN, k, 0)
            off = pl.multiple_of(k * W, 128)
            cps = []
            for tr in range(2):
                cps.append(pltpu.async_copy(
                    utt_hbm.at[pl.ds(tr * 8, 8), pl.ds(off, W)],
                    ubufs[slot][tr], sems[slot]))
                cps.append(pltpu.async_copy(
                    itt_hbm.at[pl.ds(tr * 8, 8), pl.ds(off, W)],
                    ibufs[slot][tr], sems[slot]))
            return cps

        def extract(k, bufs, mpos, mid, pend, ppos, pcnt):
            def seg_g(g, scnt):
                ids = mid[pl.ds(g * L, L)]
                mask = _w_of(ids) == k
                slots = scnt + plsc.cumsum(mask.astype(jnp.int32)) - 1
                plsc.store_scatter(
                    segpos, [slots], mpos[pl.ds(g * L, L)], mask=mask)
                plsc.store_scatter(segloc, [slots], ids - k * W, mask=mask)
                scnt = scnt + plsc.all_reduce_population_count(mask)[0]
                return jnp.minimum(scnt, SEGCAP - L)

            scnt = lax.fori_loop(0, MCAP // L, seg_g, jnp.int32(L))

            def ext_g(g, pc):
                m = scnt - L - g * L
                gmask = iota < m
                loc = jnp.where(gmask, segloc[pl.ds(L + g * L, L)], 0)
                pos = segpos[pl.ds(L + g * L, L)]
                npc = pc + jnp.minimum(jnp.maximum(m, 0), L)
                npc = jnp.minimum(npc, MCAP - L)

                slots = pc + iota

                @pl.when(m > 0)
                def _():
                    for c in range(D):
                        src = bufs[c // 8] if len(bufs) == 2 else bufs[0]
                        crow = c % 8 if len(bufs) == 2 else c
                        vals = plsc.load_gather(
                            src, [jnp.full((L,), crow, jnp.int32), loc])
                        plsc.store_scatter(
                            pend, [jnp.full((L,), c, jnp.int32), slots],
                            vals, mask=gmask)
                    plsc.store_scatter(
                        ppos, [slots], jnp.where(gmask, pos, B), mask=gmask)
                return npc

            return lax.fori_loop(0, SEGCAP // L, ext_g, pcnt)

        def drain(slot):
            for tr in range(2):
                pltpu.make_async_copy(
                    utt_hbm.at[pl.ds(tr * 8, 8), pl.ds(0, W)],
                    ubufs[slot][tr], sems[slot]).wait()
                pltpu.make_async_copy(
                    itt_hbm.at[pl.ds(tr * 8, 8), pl.ds(0, W)],
                    ibufs[slot][tr], sems[slot]).wait()

        fire(jnp.int32(0), 0)
        fire(jnp.int32(1), 1)

        def window_pair(jj, carry):
            upc, ipc = carry
            j0 = 2 * jj
            k0 = jnp.int32(wid) + 32 * j0
            k1_ = k0 + 32
            drain(0)
            upc = extract(k0, ubufs[0], umpos, umid, upend, uppos, upc)
            ipc = extract(k0, ibufs[0], impos, imid, ipend, ippos, ipc)
            fire(j0 + 2, 0)
            drain(1)
            upc = extract(k1_, ubufs[1], umpos, umid, upend, uppos, upc)
            ipc = extract(k1_, ibufs[1], impos, imid, ipend, ippos, ipc)
            fire(j0 + 3, 1)
            return upc, ipc

        upc, ipc = lax.fori_loop(
            0, NJ // 2, window_pair, (jnp.int32(0), jnp.int32(0)))
        drain(0)
        drain(1)

        pltpu.sync_copy(ut_tail_hbm, utail)
        pltpu.sync_copy(it_tail_hbm, itail)
        upc = extract(jnp.int32(NWIN), (utail,), umpos, umid,
                      upend, uppos, upc)
        ipc = extract(jnp.int32(NWIN), (itail,), impos, imid,
                      ipend, ippos, ipc)

        for ch in range(MCAP // 128):
            for g in range(128 // L):
                idxrows[ch, pl.ds(g * L, L)] = (
                    wid * (D * MCAP) + ch * 128 + g * L + iota)

        def flush(pend, ppos, rows_hbm, pmap_sp):
            cps = []
            for c in range(D):
                cps.append(pltpu.async_copy(
                    pend.at[c],
                    rows_hbm.at[pl.ds((wid * D + c) * MCAP, MCAP)], msem))
            for ch in range(MCAP // 128):
                for g in range(128 // L):
                    idxrows[8 + ch, pl.ds(g * L, L)] = (
                        ppos[pl.ds(ch * 128 + g * L, L)])
                cps.append(pltpu.async_copy(
                    idxrows.at[ch], pmap_sp.at[idxrows.at[8 + ch]], scsem))
            for cp in cps:
                cp.wait()

        flush(upend, uppos, urows_hbm, upmap_sp)
        flush(ipend, ippos, irows_hbm, ipmap_sp)

        plsc.subcore_barrier()

        @pl.when((sid == 0) & (cid == 0))
        def _():
            pltpu.sync_copy(upmap_sp, pmu0_hbm)
            pltpu.sync_copy(ipmap_sp, pmi0_hbm)

        @pl.when((sid == 0) & (cid == 1))
        def _():
            pltpu.sync_copy(upmap_sp, pmu1_hbm)
            pltpu.sync_copy(ipmap_sp, pmi1_hbm)

    return k1


@functools.lru_cache(maxsize=None)
def _build_k2(B: int):
    b_per_w = B // NW
    mesh = plsc.VectorSubcoreMesh(core_axis_name="c", subcore_axis_name="s")

    @functools.partial(
        pl.kernel,
        mesh=mesh,
        out_type=jax.ShapeDtypeStruct((B,), jnp.float32),
        compiler_params=pltpu.CompilerParams(needs_layout_passes=False),
        scratch_types=[
            pltpu.VMEM((b_per_w // 128, 128), jnp.int32),
            pltpu.VMEM((b_per_w // 128, 128), jnp.int32),
            pltpu.VMEM((b_per_w,), jnp.int32),
            pltpu.VMEM((b_per_w,), jnp.int32),
            pltpu.VMEM((b_per_w,), jnp.int32),
            pltpu.VMEM((b_per_w,), jnp.int32),
            pltpu.VMEM((D, 128), jnp.int32),
            pltpu.VMEM((D, 128), jnp.int32),
            pltpu.VMEM((D, 128), jnp.float32),
            pltpu.VMEM((D, 128), jnp.float32),
            pltpu.VMEM((b_per_w,), jnp.float32),
            pltpu.VMEM((b_per_w,), jnp.float32),
            pltpu.VMEM((b_per_w,), jnp.float32),
            pltpu.SemaphoreType.DMA,
            pltpu.SemaphoreType.DMA,
        ],
    )
    def k2(users_hbm, items_hbm, ub_hbm, ib_hbm,
           urows_hbm, irows_hbm, pmu0_hbm, pmu1_hbm, pmi0_hbm, pmi1_hbm,
           out_hbm, uidx, iidx, uslot0, uslot1, islot0, islot1,
           gidxu, gidxi, ucb, icb, ubias, ibias, outv, sem, gsem):
        wid = lax.axis_index("s") * NC + lax.axis_index("c")
        base = wid * b_per_w

        sl = pl.ds(base, b_per_w)
        pltpu.sync_copy(pmu0_hbm.at[sl], uslot0)
        pltpu.sync_copy(pmu1_hbm.at[sl], uslot1)
        pltpu.sync_copy(pmi0_hbm.at[sl], islot0)
        pltpu.sync_copy(pmi1_hbm.at[sl], islot1)

        cps = []
        for c in range(b_per_w // 128):
            pltpu.sync_copy(users_hbm.at[pl.ds(base + c * 128, 128)],
                            uidx.at[c])
            pltpu.sync_copy(items_hbm.at[pl.ds(base + c * 128, 128)],
                            iidx.at[c])
            cps.append(pltpu.async_copy(
                ub_hbm.at[uidx.at[c]], ubias.at[pl.ds(c * 128, 128)], sem))
            cps.append(pltpu.async_copy(
                ib_hbm.at[iidx.at[c]], ibias.at[pl.ds(c * 128, 128)], sem))
        for cp in cps:
            cp.wait()

        for ch in range(b_per_w // 128):
            for g in range(128 // L):
                o = pl.ds(ch * 128 + g * L, L)
                s0 = uslot0[o]
                su = jnp.where(s0 >= 0, s0, uslot1[o])
                t0 = islot0[o]
                ti = jnp.where(t0 >= 0, t0, islot1[o])
                gidxu[0, pl.ds(g * L, L)] = su
                gidxi[0, pl.ds(g * L, L)] = ti
            for c in range(1, D):
                for g in range(128 // L):
                    o = pl.ds(g * L, L)
                    gidxu[c, o] = gidxu[0, o] + c * MCAP
                    gidxi[c, o] = gidxi[0, o] + c * MCAP
            waits = []
            for c in range(D):
                waits.append(pltpu.async_copy(
                    urows_hbm.at[gidxu.at[c]], ucb.at[c], gsem))
                waits.append(pltpu.async_copy(
                    irows_hbm.at[gidxi.at[c]], icb.at[c], gsem))
            for cp in waits:
                cp.wait()
            for blk in range(128 // L):
                b = ch * (128 // L) + blk
                o = pl.ds(blk * L, L)
                acc = ubias[pl.ds(b * L, L)] + ibias[pl.ds(b * L, L)]
                for j in range(D):
                    acc = acc + ucb[j, o] * icb[j, o]
                outv[pl.ds(b * L, L)] = acc

        pltpu.sync_copy(outv, out_hbm.at[pl.ds(base, b_per_w)])

    return k2


def kernel(users, items, user_table, item_table, user_bias_table, item_bias_table):
    B = users.shape[0]
    users = users.astype(jnp.int32)
    items = items.astype(jnp.int32)
    utt = user_table.T
    itt = item_table.T
    ut_tail = jnp.pad(utt[:, COVER:], ((0, 0), (0, 128 - TAIL)))
    it_tail = jnp.pad(itt[:, COVER:], ((0, 0), (0, 128 - TAIL)))
    ub1 = user_bias_table.reshape(-1)
    ib1 = item_bias_table.reshape(-1)

    urows, irows, pmu0, pmu1, pmi0, pmi1 = _build_k1(B)(
        users, items, utt, itt, ut_tail, it_tail)
    out = _build_k2(B)(users, items, ub1, ib1,
                       urows, irows, pmu0, pmu1, pmi0, pmi1)
    return out.reshape(B, 1)
